# baseline (speedup 1.0000x reference)
import jax
import jax.numpy as jnp
from jax import lax
from jax.experimental import pallas as pl
from jax.experimental.pallas import tpu as pltpu
from jax.experimental.pallas import tpu_sc as plsc

_R, _N = 128, 32768
_L = 16
_C = _N // _L
_G = _C // _L
_T = _G // _L
_NC, _NS = 2, 16
_NW = _NC * _NS
_RPW = _R // _NW


def _sc_body(x_hbm, m_hbm, o_hbm, xv, mv, ov, sums, sg, go, off):
    wid = lax.axis_index("s") * _NC + lax.axis_index("c")
    lane = lax.iota(jnp.int32, _L)

    def do_row(r, _):
        row = wid * _RPW + r
        pltpu.sync_copy(x_hbm.at[row], xv)
        pltpu.sync_copy(m_hbm.at[row], mv)

        def p1(i, _):
            o = i * _L
            ov[pl.ds(o, _L)] = jnp.cumsum(xv[pl.ds(o, _L)] * mv[pl.ds(o, _L)])
            return 0
        lax.fori_loop(0, _C, p1, 0, unroll=8)

        def p2(g, _):
            idx = (g * _L + lane) * _L + (_L - 1)
            sums[pl.ds(g * _L, _L)] = plsc.load_gather(ov, [idx])
            return 0
        lax.fori_loop(0, _G, p2, 0, unroll=4)

        def p2b(g, _):
            sg[pl.ds(g * _L, _L)] = jnp.cumsum(sums[pl.ds(g * _L, _L)])
            return 0
        lax.fori_loop(0, _G, p2b, 0, unroll=4)

        def p3(t, carry):
            idx = (t * _L + lane) * _L + (_L - 1)
            gt = plsc.load_gather(sg, [idx])
            st = jnp.cumsum(gt)
            go[pl.ds(t * _L, _L)] = st - gt + carry
            return carry + jnp.sum(gt)
        lax.fori_loop(0, _T, p3, jnp.float32(0.0))

        def p4a(t, _):
            gov = go[pl.ds(t * _L, _L)]
            for j in range(_L):
                o = (t * _L + j) * _L
                off[pl.ds(o, _L)] = (
                    sg[pl.ds(o, _L)] - sums[pl.ds(o, _L)] + gov[j]
                )
            return 0
        lax.fori_loop(0, _T, p4a, 0)

        def p4b(g, _):
            offv = off[pl.ds(g * _L, _L)]
            for j in range(_L):
                o = (g * _L + j) * _L
                ov[pl.ds(o, _L)] = ov[pl.ds(o, _L)] + offv[j]
            return 0
        lax.fori_loop(0, _G, p4b, 0, unroll=2)

        pltpu.sync_copy(ov, o_hbm.at[row])
        return 0

    lax.fori_loop(0, _RPW, do_row, 0)


def kernel(x, mask):
    maskf = mask.astype(jnp.float32)
    f = pl.kernel(
        _sc_body,
        out_type=jax.ShapeDtypeStruct((_R, _N), jnp.float32),
        mesh=plsc.VectorSubcoreMesh(core_axis_name="c", subcore_axis_name="s"),
        scratch_types=[
            pltpu.VMEM((_N,), jnp.float32),
            pltpu.VMEM((_N,), jnp.float32),
            pltpu.VMEM((_N,), jnp.float32),
            pltpu.VMEM((_C,), jnp.float32),
            pltpu.VMEM((_C,), jnp.float32),
            pltpu.VMEM((_G,), jnp.float32),
            pltpu.VMEM((_C,), jnp.float32),
        ],
        compiler_params=pltpu.CompilerParams(needs_layout_passes=False),
    )
    return f(x, maskf)


# speedup vs baseline: 2.3390x; 2.3390x over previous
import jax
import jax.numpy as jnp
from jax import lax
from jax.experimental import pallas as pl
from jax.experimental.pallas import tpu as pltpu
from jax.experimental.pallas import tpu_sc as plsc

_R, _N = 128, 32768
_L = 16
_B = _N // 2
_C = _B // _L
_G = _C // _L
_T = _G // _L
_NC, _NS = 2, 16
_NW = _NC * _NS
_RPW = _R // _NW
_NB = _RPW * 2


def _sc_body(x_hbm, m_hbm, o_hbm, xv, mv, ov, sums, sg, go, off, sems):
    wid = lax.axis_index("s") * _NC + lax.axis_index("c")
    lane = lax.iota(jnp.int32, _L)

    def src(hbm, b):
        row = wid * _RPW + b // 2
        return hbm.at[row, pl.ds((b % 2) * _B, _B)]

    def start_in(b):
        p = b % 2
        return (
            pltpu.async_copy(src(x_hbm, b), xv.at[p], sems.at[p]),
            pltpu.async_copy(src(m_hbm, b), mv.at[p], sems.at[2 + p]),
        )

    def compute_block(p, base):
        @plsc.parallel_loop(0, _C, unroll=8)
        def _(i):
            o = i * _L
            ov[p, pl.ds(o, _L)] = jnp.cumsum(
                xv[p, pl.ds(o, _L)] * mv[p, pl.ds(o, _L)]
            )

        pidx = jnp.full((_L,), p, jnp.int32)

        @plsc.parallel_loop(0, _G, unroll=4)
        def _(g):
            idx = (g * _L + lane) * _L + (_L - 1)
            sums[pl.ds(g * _L, _L)] = plsc.load_gather(ov, [pidx, idx])

        @plsc.parallel_loop(0, _G, unroll=4)
        def _(g):
            sg[pl.ds(g * _L, _L)] = jnp.cumsum(sums[pl.ds(g * _L, _L)])

        def p3(t, carry):
            idx = (t * _L + lane) * _L + (_L - 1)
            gt = plsc.load_gather(sg, [idx])
            st = jnp.cumsum(gt)
            go[pl.ds(t * _L, _L)] = st - gt + carry
            return carry + jnp.sum(gt)
        total = lax.fori_loop(0, _T, p3, base)

        @plsc.parallel_loop(0, _T)
        def _(t):
            gov = go[pl.ds(t * _L, _L)]
            for j in range(_L):
                o = (t * _L + j) * _L
                off[pl.ds(o, _L)] = (
                    sg[pl.ds(o, _L)] - sums[pl.ds(o, _L)] + gov[j]
                )

        @plsc.parallel_loop(0, _G, unroll=2)
        def _(g):
            offv = off[pl.ds(g * _L, _L)]
            for j in range(_L):
                o = (g * _L + j) * _L
                ov[p, pl.ds(o, _L)] = ov[p, pl.ds(o, _L)] + offv[j]

        return total

    in_cps = {0: start_in(0), 1: start_in(1)}
    out_cps = {}
    base = jnp.float32(0.0)
    for b in range(_NB):
        p = b % 2
        if b >= 2:
            out_cps.pop(b - 2).wait()
        for cp in in_cps.pop(b):
            cp.wait()
        if b % 2 == 0:
            base = jnp.float32(0.0)
        base = compute_block(p, base)
        if b + 2 < _NB:
            in_cps[b + 2] = start_in(b + 2)
        out_cps[b] = pltpu.async_copy(ov.at[p], src(o_hbm, b), sems.at[4 + p])
    for cp in out_cps.values():
        cp.wait()


def kernel(x, mask):
    maskf = mask.astype(jnp.float32)
    f = pl.kernel(
        _sc_body,
        out_type=jax.ShapeDtypeStruct((_R, _N), jnp.float32),
        mesh=plsc.VectorSubcoreMesh(core_axis_name="c", subcore_axis_name="s"),
        scratch_types=[
            pltpu.VMEM((2, _B), jnp.float32),
            pltpu.VMEM((2, _B), jnp.float32),
            pltpu.VMEM((2, _B), jnp.float32),
            pltpu.VMEM((_C,), jnp.float32),
            pltpu.VMEM((_C,), jnp.float32),
            pltpu.VMEM((_G,), jnp.float32),
            pltpu.VMEM((_C,), jnp.float32),
            pltpu.SemaphoreType.DMA((6,)),
        ],
        compiler_params=pltpu.CompilerParams(needs_layout_passes=False),
    )
    return f(x, maskf)
